# baseline (speedup 1.0000x reference)
import jax
import jax.numpy as jnp
from jax.experimental import pallas as pl

N = 100000
D_NODE = 128
WARM_DIM = 64
BLOCK = 20000
NUM_BLOCKS = N // BLOCK
CHUNK = 4000
NCHUNK = BLOCK // CHUNK


def _fused_body(t_ref, x_ref, wmu_ref, bmu_ref, wlv_ref, blv_ref,
                wdec_ref, bdec_ref, out_ref, kl_ref):
    i = pl.program_id(0)

    def chunk_step(c, carry):
        kl_acc, cnt = carry
        sl = pl.ds(c * CHUNK, CHUNK)
        x = x_ref[sl, :]
        warm_col = (t_ref[sl, :] == 1).astype(jnp.float32)

        mu = jnp.dot(x, wmu_ref[...], preferred_element_type=jnp.float32) + bmu_ref[...]
        logvar = jnp.dot(x, wlv_ref[...], preferred_element_type=jnp.float32) + blv_ref[...]
        dec = jnp.dot(mu, wdec_ref[...], preferred_element_type=jnp.float32) + bdec_ref[...]

        out_ref[sl, :] = x + warm_col * (dec - x)

        kl_terms = 1.0 + logvar - mu * mu - jnp.exp(logvar)
        kl_part = jnp.sum(warm_col * kl_terms)
        return kl_acc + kl_part, cnt + jnp.sum(warm_col)

    kl_sum, cnt = jax.lax.fori_loop(0, NCHUNK, chunk_step, (0.0, 0.0))

    lane = jax.lax.broadcasted_iota(jnp.int32, (1, 128), 1)
    row = jnp.where(lane == 0, kl_sum, 0.0) + jnp.where(lane == 1, cnt, 0.0)

    @pl.when(i == 0)
    def _init():
        kl_ref[...] = row

    @pl.when(i > 0)
    def _acc():
        kl_ref[...] += row


def kernel(node_features, node_tiers, W_mu, b_mu, W_logvar, b_logvar, W_dec, b_dec):
    tiers_col = node_tiers.astype(jnp.int8).reshape(N, 1)

    grid = (NUM_BLOCKS,)
    out_shapes = (
        jax.ShapeDtypeStruct((N, D_NODE), jnp.float32),
        jax.ShapeDtypeStruct((1, 128), jnp.float32),
    )
    new_features, kl_stats = pl.pallas_call(
        _fused_body,
        grid=grid,
        in_specs=[
            pl.BlockSpec((BLOCK, 1), lambda i: (i, 0)),
            pl.BlockSpec((BLOCK, D_NODE), lambda i: (i, 0)),
            pl.BlockSpec((D_NODE, WARM_DIM), lambda i: (0, 0)),
            pl.BlockSpec((WARM_DIM,), lambda i: (0,)),
            pl.BlockSpec((D_NODE, WARM_DIM), lambda i: (0, 0)),
            pl.BlockSpec((WARM_DIM,), lambda i: (0,)),
            pl.BlockSpec((WARM_DIM, D_NODE), lambda i: (0, 0)),
            pl.BlockSpec((D_NODE,), lambda i: (0,)),
        ],
        out_specs=(
            pl.BlockSpec((BLOCK, D_NODE), lambda i: (i, 0)),
            pl.BlockSpec((1, 128), lambda i: (0, 0)),
        ),
        out_shape=out_shapes,
    )(tiers_col, node_features, W_mu, b_mu, W_logvar, b_logvar, W_dec, b_dec)

    kl_sum = kl_stats[0, 0]
    n_warm_elems = kl_stats[0, 1] * WARM_DIM
    kl_loss = -0.5 * (kl_sum / n_warm_elems)
    return new_features, kl_loss


# speedup vs baseline: 1.0039x; 1.0039x over previous
import jax
import jax.numpy as jnp
from jax.experimental import pallas as pl

N = 100000
D_NODE = 128
WARM_DIM = 64
BLOCK = 20000
NUM_BLOCKS = N // BLOCK
CHUNK = 4000
NCHUNK = BLOCK // CHUNK


def _fused_body(t_ref, x_ref, wmu_ref, bmu_ref, wlv_ref, blv_ref,
                wdec_ref, bdec_ref, out_ref, kl_ref):
    i = pl.program_id(0)

    kl_sum = 0.0
    cnt = 0.0
    for c in range(NCHUNK):
        sl = pl.ds(c * CHUNK, CHUNK)
        x = x_ref[sl, :]
        warm_col = (t_ref[sl, :] == 1).astype(jnp.float32)

        mu = jnp.dot(x, wmu_ref[...], preferred_element_type=jnp.float32) + bmu_ref[...]
        logvar = jnp.dot(x, wlv_ref[...], preferred_element_type=jnp.float32) + blv_ref[...]
        dec = jnp.dot(mu, wdec_ref[...], preferred_element_type=jnp.float32) + bdec_ref[...]

        out_ref[sl, :] = x + warm_col * (dec - x)

        kl_terms = 1.0 + logvar - mu * mu - jnp.exp(logvar)
        kl_sum = kl_sum + jnp.sum(warm_col * kl_terms)
        cnt = cnt + jnp.sum(warm_col)

    lane = jax.lax.broadcasted_iota(jnp.int32, (1, 128), 1)
    row = jnp.where(lane == 0, kl_sum, 0.0) + jnp.where(lane == 1, cnt, 0.0)

    @pl.when(i == 0)
    def _init():
        kl_ref[...] = row

    @pl.when(i > 0)
    def _acc():
        kl_ref[...] += row


def kernel(node_features, node_tiers, W_mu, b_mu, W_logvar, b_logvar, W_dec, b_dec):
    tiers_col = node_tiers.astype(jnp.int8).reshape(N, 1)

    grid = (NUM_BLOCKS,)
    out_shapes = (
        jax.ShapeDtypeStruct((N, D_NODE), jnp.float32),
        jax.ShapeDtypeStruct((1, 128), jnp.float32),
    )
    new_features, kl_stats = pl.pallas_call(
        _fused_body,
        grid=grid,
        in_specs=[
            pl.BlockSpec((BLOCK, 1), lambda i: (i, 0)),
            pl.BlockSpec((BLOCK, D_NODE), lambda i: (i, 0)),
            pl.BlockSpec((D_NODE, WARM_DIM), lambda i: (0, 0)),
            pl.BlockSpec((WARM_DIM,), lambda i: (0,)),
            pl.BlockSpec((D_NODE, WARM_DIM), lambda i: (0, 0)),
            pl.BlockSpec((WARM_DIM,), lambda i: (0,)),
            pl.BlockSpec((WARM_DIM, D_NODE), lambda i: (0, 0)),
            pl.BlockSpec((D_NODE,), lambda i: (0,)),
        ],
        out_specs=(
            pl.BlockSpec((BLOCK, D_NODE), lambda i: (i, 0)),
            pl.BlockSpec((1, 128), lambda i: (0, 0)),
        ),
        out_shape=out_shapes,
    )(tiers_col, node_features, W_mu, b_mu, W_logvar, b_logvar, W_dec, b_dec)

    kl_sum = kl_stats[0, 0]
    n_warm_elems = kl_stats[0, 1] * WARM_DIM
    kl_loss = -0.5 * (kl_sum / n_warm_elems)
    return new_features, kl_loss


# speedup vs baseline: 1.1004x; 1.0962x over previous
import jax
import jax.numpy as jnp
from jax.experimental import pallas as pl

N = 100000
D_NODE = 128
WARM_DIM = 64
BLOCK = 10000
NUM_BLOCKS = N // BLOCK


def _fused_body(t_ref, x_ref, wmu_ref, bmu_ref, wlv_ref, blv_ref,
                wdec_ref, bdec_ref, out_ref, kl_ref):
    i = pl.program_id(0)
    x = x_ref[...]
    warm_col = (t_ref[...] == 1).astype(jnp.float32)

    mu = jnp.dot(x, wmu_ref[...], preferred_element_type=jnp.float32) + bmu_ref[...]
    logvar = jnp.dot(x, wlv_ref[...], preferred_element_type=jnp.float32) + blv_ref[...]
    dec = jnp.dot(mu, wdec_ref[...], preferred_element_type=jnp.float32) + bdec_ref[...]

    out_ref[...] = x + warm_col * (dec - x)

    kl_terms = 1.0 + logvar - mu * mu - jnp.exp(logvar)
    partial = jnp.sum(warm_col * kl_terms)
    cnt = jnp.sum(warm_col)

    lane = jax.lax.broadcasted_iota(jnp.int32, (1, 128), 1)
    row = jnp.where(lane == 0, partial, 0.0) + jnp.where(lane == 1, cnt, 0.0)

    @pl.when(i == 0)
    def _init():
        kl_ref[...] = row

    @pl.when(i > 0)
    def _acc():
        kl_ref[...] += row


def kernel(node_features, node_tiers, W_mu, b_mu, W_logvar, b_logvar, W_dec, b_dec):
    tiers_col = node_tiers.astype(jnp.int16).reshape(N, 1)

    grid = (NUM_BLOCKS,)
    out_shapes = (
        jax.ShapeDtypeStruct((N, D_NODE), jnp.float32),
        jax.ShapeDtypeStruct((1, 128), jnp.float32),
    )
    new_features, kl_stats = pl.pallas_call(
        _fused_body,
        grid=grid,
        in_specs=[
            pl.BlockSpec((BLOCK, 1), lambda i: (i, 0)),
            pl.BlockSpec((BLOCK, D_NODE), lambda i: (i, 0)),
            pl.BlockSpec((D_NODE, WARM_DIM), lambda i: (0, 0)),
            pl.BlockSpec((WARM_DIM,), lambda i: (0,)),
            pl.BlockSpec((D_NODE, WARM_DIM), lambda i: (0, 0)),
            pl.BlockSpec((WARM_DIM,), lambda i: (0,)),
            pl.BlockSpec((WARM_DIM, D_NODE), lambda i: (0, 0)),
            pl.BlockSpec((D_NODE,), lambda i: (0,)),
        ],
        out_specs=(
            pl.BlockSpec((BLOCK, D_NODE), lambda i: (i, 0)),
            pl.BlockSpec((1, 128), lambda i: (0, 0)),
        ),
        out_shape=out_shapes,
    )(tiers_col, node_features, W_mu, b_mu, W_logvar, b_logvar, W_dec, b_dec)

    kl_sum = kl_stats[0, 0]
    n_warm_elems = kl_stats[0, 1] * WARM_DIM
    kl_loss = -0.5 * (kl_sum / n_warm_elems)
    return new_features, kl_loss


# speedup vs baseline: 1.1088x; 1.0076x over previous
import jax
import jax.numpy as jnp
from jax.experimental import pallas as pl
from jax.experimental.pallas import tpu as pltpu

N = 100000
D_NODE = 128
WARM_DIM = 64
BLOCK = 4000
NUM_BLOCKS = N // BLOCK
NBUF = 4


def _fused_body(t_ref, x_hbm, wmu_ref, bmu_ref, wlv_ref, blv_ref,
                wdec_ref, bdec_ref, out_hbm, kl_ref,
                xbuf, obuf, insem, outsem):
    i = pl.program_id(0)

    def incopy(blk, slot):
        return pltpu.make_async_copy(
            x_hbm.at[pl.ds(blk * BLOCK, BLOCK), :], xbuf.at[slot],
            insem.at[slot])

    def outcopy(blk, slot):
        return pltpu.make_async_copy(
            obuf.at[slot], out_hbm.at[pl.ds(blk * BLOCK, BLOCK), :],
            outsem.at[slot])

    @pl.when(i == 0)
    def _prologue():
        for s in range(NBUF):
            incopy(s, s).start()

    s = jax.lax.rem(i, NBUF)
    incopy(i, s).wait()

    @pl.when(i >= NBUF)
    def _drain_prev():
        outcopy(i - NBUF, s).wait()

    x = xbuf[s]
    warm_col = (t_ref[...] == 1).astype(jnp.float32)

    mu = jnp.dot(x, wmu_ref[...], preferred_element_type=jnp.float32) + bmu_ref[...]
    logvar = jnp.dot(x, wlv_ref[...], preferred_element_type=jnp.float32) + blv_ref[...]
    dec = jnp.dot(mu, wdec_ref[...], preferred_element_type=jnp.float32) + bdec_ref[...]

    obuf[s] = x + warm_col * (dec - x)
    outcopy(i, s).start()

    @pl.when(i + NBUF < NUM_BLOCKS)
    def _prefetch():
        incopy(i + NBUF, s).start()

    kl_terms = 1.0 + logvar - mu * mu - jnp.exp(logvar)
    partial = jnp.sum(warm_col * kl_terms)
    cnt = jnp.sum(warm_col)

    lane = jax.lax.broadcasted_iota(jnp.int32, (1, 128), 1)
    row = jnp.where(lane == 0, partial, 0.0) + jnp.where(lane == 1, cnt, 0.0)

    @pl.when(i == 0)
    def _init():
        kl_ref[...] = row

    @pl.when(i > 0)
    def _acc():
        kl_ref[...] += row

    @pl.when(i == NUM_BLOCKS - 1)
    def _epilogue():
        for d in range(NBUF):
            blk = NUM_BLOCKS - NBUF + d
            outcopy(blk, blk % NBUF).wait()


def kernel(node_features, node_tiers, W_mu, b_mu, W_logvar, b_logvar, W_dec, b_dec):
    tiers_col = node_tiers.astype(jnp.int32).reshape(N, 1)

    grid = (NUM_BLOCKS,)
    out_shapes = (
        jax.ShapeDtypeStruct((N, D_NODE), jnp.float32),
        jax.ShapeDtypeStruct((1, 128), jnp.float32),
    )
    new_features, kl_stats = pl.pallas_call(
        _fused_body,
        grid=grid,
        in_specs=[
            pl.BlockSpec((BLOCK, 1), lambda i: (i, 0)),
            pl.BlockSpec(memory_space=pltpu.MemorySpace.HBM),
            pl.BlockSpec((D_NODE, WARM_DIM), lambda i: (0, 0)),
            pl.BlockSpec((WARM_DIM,), lambda i: (0,)),
            pl.BlockSpec((D_NODE, WARM_DIM), lambda i: (0, 0)),
            pl.BlockSpec((WARM_DIM,), lambda i: (0,)),
            pl.BlockSpec((WARM_DIM, D_NODE), lambda i: (0, 0)),
            pl.BlockSpec((D_NODE,), lambda i: (0,)),
        ],
        out_specs=(
            pl.BlockSpec(memory_space=pltpu.MemorySpace.HBM),
            pl.BlockSpec((1, 128), lambda i: (0, 0)),
        ),
        out_shape=out_shapes,
        scratch_shapes=[
            pltpu.MemorySpace.VMEM((NBUF, BLOCK, D_NODE), jnp.float32),
            pltpu.MemorySpace.VMEM((NBUF, BLOCK, D_NODE), jnp.float32),
            pltpu.SemaphoreType.DMA((NBUF,)),
            pltpu.SemaphoreType.DMA((NBUF,)),
        ],
    )(tiers_col, node_features, W_mu, b_mu, W_logvar, b_logvar, W_dec, b_dec)

    kl_sum = kl_stats[0, 0]
    n_warm_elems = kl_stats[0, 1] * WARM_DIM
    kl_loss = -0.5 * (kl_sum / n_warm_elems)
    return new_features, kl_loss


# speedup vs baseline: 1.1818x; 1.0658x over previous
import jax
import jax.numpy as jnp
from jax.experimental import pallas as pl
from jax.experimental.pallas import tpu as pltpu

N = 100000
D_NODE = 128
WARM_DIM = 64
BLOCK = 10000
NUM_BLOCKS = N // BLOCK
NBUF = 3


def _fused_body(t_ref, x_hbm, wmu_ref, bmu_ref, wlv_ref, blv_ref,
                wdec_ref, bdec_ref, out_hbm, kl_ref,
                xbuf, obuf, insem, outsem):
    i = pl.program_id(0)

    def incopy(blk, slot):
        return pltpu.make_async_copy(
            x_hbm.at[pl.ds(blk * BLOCK, BLOCK), :], xbuf.at[slot],
            insem.at[slot])

    def outcopy(blk, slot):
        return pltpu.make_async_copy(
            obuf.at[slot], out_hbm.at[pl.ds(blk * BLOCK, BLOCK), :],
            outsem.at[slot])

    @pl.when(i == 0)
    def _prologue():
        for s in range(NBUF):
            incopy(s, s).start()

    s = jax.lax.rem(i, NBUF)
    incopy(i, s).wait()

    @pl.when(i >= NBUF)
    def _drain_prev():
        outcopy(i - NBUF, s).wait()

    x = xbuf[s]
    warm_col = (t_ref[...] == 1).astype(jnp.float32)

    mu = jnp.dot(x, wmu_ref[...], preferred_element_type=jnp.float32) + bmu_ref[...]
    logvar = jnp.dot(x, wlv_ref[...], preferred_element_type=jnp.float32) + blv_ref[...]
    dec = jnp.dot(mu, wdec_ref[...], preferred_element_type=jnp.float32) + bdec_ref[...]

    obuf[s] = x + warm_col * (dec - x)
    outcopy(i, s).start()

    @pl.when(i + NBUF < NUM_BLOCKS)
    def _prefetch():
        incopy(i + NBUF, s).start()

    kl_terms = 1.0 + logvar - mu * mu - jnp.exp(logvar)
    partial = jnp.sum(warm_col * kl_terms)
    cnt = jnp.sum(warm_col)

    lane = jax.lax.broadcasted_iota(jnp.int32, (1, 128), 1)
    row = jnp.where(lane == 0, partial, 0.0) + jnp.where(lane == 1, cnt, 0.0)

    @pl.when(i == 0)
    def _init():
        kl_ref[...] = row

    @pl.when(i > 0)
    def _acc():
        kl_ref[...] += row

    @pl.when(i == NUM_BLOCKS - 1)
    def _epilogue():
        for d in range(NBUF):
            blk = NUM_BLOCKS - NBUF + d
            outcopy(blk, blk % NBUF).wait()


def kernel(node_features, node_tiers, W_mu, b_mu, W_logvar, b_logvar, W_dec, b_dec):
    tiers_col = node_tiers.astype(jnp.int32).reshape(N, 1)

    grid = (NUM_BLOCKS,)
    out_shapes = (
        jax.ShapeDtypeStruct((N, D_NODE), jnp.float32),
        jax.ShapeDtypeStruct((1, 128), jnp.float32),
    )
    new_features, kl_stats = pl.pallas_call(
        _fused_body,
        grid=grid,
        in_specs=[
            pl.BlockSpec((BLOCK, 1), lambda i: (i, 0)),
            pl.BlockSpec(memory_space=pltpu.MemorySpace.HBM),
            pl.BlockSpec((D_NODE, WARM_DIM), lambda i: (0, 0)),
            pl.BlockSpec((WARM_DIM,), lambda i: (0,)),
            pl.BlockSpec((D_NODE, WARM_DIM), lambda i: (0, 0)),
            pl.BlockSpec((WARM_DIM,), lambda i: (0,)),
            pl.BlockSpec((WARM_DIM, D_NODE), lambda i: (0, 0)),
            pl.BlockSpec((D_NODE,), lambda i: (0,)),
        ],
        out_specs=(
            pl.BlockSpec(memory_space=pltpu.MemorySpace.HBM),
            pl.BlockSpec((1, 128), lambda i: (0, 0)),
        ),
        out_shape=out_shapes,
        scratch_shapes=[
            pltpu.MemorySpace.VMEM((NBUF, BLOCK, D_NODE), jnp.float32),
            pltpu.MemorySpace.VMEM((NBUF, BLOCK, D_NODE), jnp.float32),
            pltpu.SemaphoreType.DMA((NBUF,)),
            pltpu.SemaphoreType.DMA((NBUF,)),
        ],
    )(tiers_col, node_features, W_mu, b_mu, W_logvar, b_logvar, W_dec, b_dec)

    kl_sum = kl_stats[0, 0]
    n_warm_elems = kl_stats[0, 1] * WARM_DIM
    kl_loss = -0.5 * (kl_sum / n_warm_elems)
    return new_features, kl_loss
